# baseline (speedup 1.0000x reference)
import functools

import jax
import jax.numpy as jnp
from jax import lax
from jax.experimental import pallas as pl
from jax.experimental.pallas import tpu as pltpu
from jax.experimental.pallas import tpu_sc as plsc

B = 16384
W = 20
K = 20
D = 64
NW = 32
ROWS_PER_W = B // NW
C = 16
NCH = ROWS_PER_W // C
S = 32


def _sc_scores(emb_in, emb_out, ctx1d, tgt, neg1d):
    mesh = plsc.VectorSubcoreMesh(core_axis_name="c", subcore_axis_name="s")

    buf_types = []
    for _ in range(2):
        buf_types += [
            pltpu.VMEM((C * W,), jnp.int32),
            pltpu.VMEM((C * K,), jnp.int32),
            pltpu.VMEM((C,), jnp.int32),
            pltpu.VMEM((C * W, D), jnp.float32),
            pltpu.VMEM((C * K, D), jnp.float32),
            pltpu.VMEM((C, D), jnp.float32),
            pltpu.VMEM((C * S,), jnp.float32),
        ]
    sem_types = [pltpu.SemaphoreType.DMA] * 6

    @functools.partial(
        pl.kernel,
        out_type=jax.ShapeDtypeStruct((B * S,), jnp.float32),
        mesh=mesh,
        scratch_types=buf_types + sem_types,
        compiler_params=pltpu.CompilerParams(use_tc_tiling_on_sc=False),
    )
    def scores_kernel(emb_in_hbm, emb_out_hbm, ctx_hbm, tgt_hbm, neg_hbm,
                      sc_hbm, *scratch):
        bufs = [scratch[0:7], scratch[7:14]]
        sem_idx = scratch[14:16]
        sem_g = scratch[16:18]
        sem_o = scratch[18:20]

        cid = lax.axis_index("c")
        sid = lax.axis_index("s")
        wid = sid * 2 + cid
        wbase = wid * ROWS_PER_W

        lane = lax.iota(jnp.int32, 16)
        perms = [lane ^ s for s in (8, 4, 2, 1)]
        zeros = jnp.zeros((16,), jnp.float32)
        gdn = lax.GatherDimensionNumbers(
            offset_dims=(), collapsed_slice_dims=(0,), start_index_map=(0,))

        def lanesum(v):
            for p16 in perms:
                v = v + lax.gather(
                    v, p16[:, None], gdn, (1,),
                    mode=lax.GatherScatterMode.PROMISE_IN_BOUNDS)
            return v

        def issue_idx(ch, b):
            ctx_idx, neg_idx, tgt_idx = bufs[b][0:3]
            base = wbase + ch * C
            ibase = base * W
            pltpu.async_copy(ctx_hbm.at[pl.ds(ibase, C * W)], ctx_idx,
                             sem_idx[b])
            pltpu.async_copy(neg_hbm.at[pl.ds(ibase, C * K)], neg_idx,
                             sem_idx[b])
            pltpu.async_copy(tgt_hbm.at[pl.ds(base, C)], tgt_idx, sem_idx[b])

        def drain_idx(b):
            ctx_idx, neg_idx, tgt_idx = bufs[b][0:3]
            pltpu.make_async_copy(ctx_hbm.at[pl.ds(0, C * W)], ctx_idx,
                                  sem_idx[b]).wait()
            pltpu.make_async_copy(neg_hbm.at[pl.ds(0, C * K)], neg_idx,
                                  sem_idx[b]).wait()
            pltpu.make_async_copy(tgt_hbm.at[pl.ds(0, C)], tgt_idx,
                                  sem_idx[b]).wait()

        def issue_gathers(b):
            ctx_idx, neg_idx, tgt_idx, ctx_rows, neg_rows, tgt_rows = \
                bufs[b][0:6]
            pltpu.async_copy(emb_in_hbm.at[ctx_idx], ctx_rows, sem_g[b])
            pltpu.async_copy(emb_out_hbm.at[neg_idx], neg_rows, sem_g[b])
            pltpu.async_copy(emb_out_hbm.at[tgt_idx], tgt_rows, sem_g[b])

        def drain_gathers(b):
            ctx_idx, neg_idx, tgt_idx, ctx_rows, neg_rows, tgt_rows = \
                bufs[b][0:6]
            pltpu.make_async_copy(emb_in_hbm.at[ctx_idx], ctx_rows,
                                  sem_g[b]).wait()
            pltpu.make_async_copy(emb_out_hbm.at[neg_idx], neg_rows,
                                  sem_g[b]).wait()
            pltpu.make_async_copy(emb_out_hbm.at[tgt_idx], tgt_rows,
                                  sem_g[b]).wait()

        def drain_scores(b):
            scores_v = bufs[b][6]
            pltpu.make_async_copy(scores_v, sc_hbm.at[pl.ds(0, C * S)],
                                  sem_o[b]).wait()

        def compute(ch, b):
            ctx_rows, neg_rows, tgt_rows, scores_v = bufs[b][3:7]
            base = wbase + ch * C

            def row_body(i, rc):
                rb = i * W
                acc = [ctx_rows[rb, pl.ds(c * 16, 16)] for c in range(4)]
                for j in range(1, W):
                    for c in range(4):
                        acc[c] = acc[c] + ctx_rows[rb + j, pl.ds(c * 16, 16)]
                nv0 = zeros
                nv1 = zeros
                p = acc[0] * tgt_rows[i, pl.ds(0, 16)]
                for c in range(1, 4):
                    p = p + acc[c] * tgt_rows[i, pl.ds(c * 16, 16)]
                nv1 = jnp.where(lane == (K - 16), lanesum(p), nv1)
                for k in range(K):
                    q = acc[0] * neg_rows[rb + k, pl.ds(0, 16)]
                    for c in range(1, 4):
                        q = q + acc[c] * neg_rows[rb + k, pl.ds(c * 16, 16)]
                    cq = lanesum(q)
                    if k < 16:
                        nv0 = jnp.where(lane == k, cq, nv0)
                    else:
                        nv1 = jnp.where(lane == (k - 16), cq, nv1)
                scores_v[pl.ds(i * S, 16)] = nv0
                scores_v[pl.ds(i * S + 16, 16)] = nv1
                return rc

            drain_scores(b)
            lax.fori_loop(0, C, row_body, 0)
            pltpu.async_copy(scores_v, sc_hbm.at[pl.ds(base * S, C * S)],
                             sem_o[b])

        issue_idx(jnp.int32(0), 0)
        drain_idx(0)
        issue_gathers(0)
        issue_idx(jnp.int32(1), 1)
        pltpu.async_copy(bufs[0][6], sc_hbm.at[pl.ds(wbase * S, C * S)],
                         sem_o[0])
        pltpu.async_copy(bufs[1][6], sc_hbm.at[pl.ds((wbase + C) * S, C * S)],
                         sem_o[1])

        last = jnp.int32(NCH - 1)

        def pair_body(i, carry):
            ch0 = i * 2
            ch1 = ch0 + 1
            drain_idx(1)
            issue_gathers(1)
            drain_gathers(0)
            issue_idx(jnp.minimum(ch0 + 2, last), 0)
            compute(ch0, 0)
            drain_idx(0)
            issue_gathers(0)
            drain_gathers(1)
            issue_idx(jnp.minimum(ch1 + 2, last), 1)
            compute(ch1, 1)
            return carry

        lax.fori_loop(0, NCH // 2, pair_body, 0)

        drain_gathers(0)
        drain_idx(1)
        drain_scores(0)
        drain_scores(1)

    return scores_kernel(emb_in, emb_out, ctx1d, tgt, neg1d)


def _transpose_body(x_ref, o_ref):
    o_ref[...] = x_ref[...].T


def _tc_transpose(xt, bn):
    v = xt.shape[1]
    grid = (v + bn - 1) // bn
    return pl.pallas_call(
        _transpose_body,
        grid=(grid,),
        in_specs=[pl.BlockSpec((D, bn), lambda i: (0, i))],
        out_specs=pl.BlockSpec((bn, D), lambda i: (i, 0)),
        out_shape=jax.ShapeDtypeStruct((v, D), jnp.float32),
    )(xt)


def _loss_body(sc_ref, out_ref):
    x = sc_ref[...]
    l32 = lax.broadcasted_iota(jnp.int32, x.shape, 1) % S
    soft_abs = jnp.log1p(jnp.exp(-jnp.abs(x)))
    sp_pos = jnp.maximum(x, 0.0) + soft_abs
    sp_neg = jnp.maximum(-x, 0.0) + soft_abs
    n_sum = jnp.sum(jnp.where(l32 < K, sp_pos, 0.0))
    t_sum = jnp.sum(jnp.where(l32 == K, sp_neg, 0.0))
    out_ref[0, 0] = t_sum / B + n_sum / (B * K)


def kernel(context, target, negative_samples, emb_in, emb_out):
    ctx1d = context.reshape(B * W).astype(jnp.int32)
    neg1d = negative_samples.reshape(B * K).astype(jnp.int32)
    tgt = target.astype(jnp.int32)

    emb_in_rm = _tc_transpose(emb_in.T, 2048)
    emb_out_rm = _tc_transpose(emb_out.T, 2048)

    scores = _sc_scores(emb_in_rm, emb_out_rm, ctx1d, tgt, neg1d)

    loss = pl.pallas_call(
        _loss_body,
        out_shape=jax.ShapeDtypeStruct((1, 1), jnp.float32),
        in_specs=[pl.BlockSpec(memory_space=pltpu.VMEM)],
        out_specs=pl.BlockSpec(memory_space=pltpu.SMEM),
    )(scores.reshape(B * S // 128, 128))
    return loss[0, 0]


# speedup vs baseline: 1.5292x; 1.5292x over previous
import functools

import jax
import jax.numpy as jnp
from jax import lax
from jax.experimental import pallas as pl
from jax.experimental.pallas import tpu as pltpu
from jax.experimental.pallas import tpu_sc as plsc

B = 16384
W = 20
K = 20
D = 64
NW = 32
ROWS_PER_W = B // NW
C = 16
NCH = ROWS_PER_W // C
S = 32


def _sc_scores(emb_in, emb_out, ctx1d, tgt, neg1d):
    mesh = plsc.VectorSubcoreMesh(core_axis_name="c", subcore_axis_name="s")

    buf_types = []
    for _ in range(2):
        buf_types += [
            pltpu.VMEM((C * W,), jnp.int32),
            pltpu.VMEM((C * K,), jnp.int32),
            pltpu.VMEM((C,), jnp.int32),
            pltpu.VMEM((C * W, D), jnp.float32),
            pltpu.VMEM((C * K, D), jnp.float32),
            pltpu.VMEM((C, D), jnp.float32),
            pltpu.VMEM((C * S,), jnp.float32),
        ]
    sem_types = [pltpu.SemaphoreType.DMA] * 6

    @functools.partial(
        pl.kernel,
        out_type=jax.ShapeDtypeStruct((B * S,), jnp.float32),
        mesh=mesh,
        scratch_types=buf_types + sem_types,
        compiler_params=pltpu.CompilerParams(use_tc_tiling_on_sc=False),
    )
    def scores_kernel(emb_in_hbm, emb_out_hbm, ctx_hbm, tgt_hbm, neg_hbm,
                      sc_hbm, *scratch):
        bufs = [scratch[0:7], scratch[7:14]]
        sem_idx = scratch[14:16]
        sem_g = scratch[16:18]
        sem_o = scratch[18:20]

        cid = lax.axis_index("c")
        sid = lax.axis_index("s")
        wid = sid * 2 + cid
        wbase = wid * ROWS_PER_W

        lane = lax.iota(jnp.int32, 16)
        perms = [lane ^ s for s in (8, 4, 2, 1)]
        zeros = jnp.zeros((16,), jnp.float32)
        gdn = lax.GatherDimensionNumbers(
            offset_dims=(), collapsed_slice_dims=(0,), start_index_map=(0,))

        def lanesum(v):
            for p16 in perms:
                v = v + lax.gather(
                    v, p16[:, None], gdn, (1,),
                    mode=lax.GatherScatterMode.PROMISE_IN_BOUNDS)
            return v

        def issue_idx(ch, b):
            ctx_idx, neg_idx, tgt_idx = bufs[b][0:3]
            base = wbase + ch * C
            ibase = base * W
            pltpu.async_copy(ctx_hbm.at[pl.ds(ibase, C * W)], ctx_idx,
                             sem_idx[b])
            pltpu.async_copy(neg_hbm.at[pl.ds(ibase, C * K)], neg_idx,
                             sem_idx[b])
            pltpu.async_copy(tgt_hbm.at[pl.ds(base, C)], tgt_idx, sem_idx[b])

        def drain_idx(b):
            ctx_idx, neg_idx, tgt_idx = bufs[b][0:3]
            pltpu.make_async_copy(ctx_hbm.at[pl.ds(0, C * W)], ctx_idx,
                                  sem_idx[b]).wait()
            pltpu.make_async_copy(neg_hbm.at[pl.ds(0, C * K)], neg_idx,
                                  sem_idx[b]).wait()
            pltpu.make_async_copy(tgt_hbm.at[pl.ds(0, C)], tgt_idx,
                                  sem_idx[b]).wait()

        def issue_gathers(b):
            ctx_idx, neg_idx, tgt_idx, ctx_rows, neg_rows, tgt_rows = \
                bufs[b][0:6]
            pltpu.async_copy(emb_in_hbm.at[ctx_idx], ctx_rows, sem_g[b])
            pltpu.async_copy(emb_out_hbm.at[neg_idx], neg_rows, sem_g[b])
            pltpu.async_copy(emb_out_hbm.at[tgt_idx], tgt_rows, sem_g[b])

        def drain_gathers(b):
            ctx_idx, neg_idx, tgt_idx, ctx_rows, neg_rows, tgt_rows = \
                bufs[b][0:6]
            pltpu.make_async_copy(emb_in_hbm.at[ctx_idx], ctx_rows,
                                  sem_g[b]).wait()
            pltpu.make_async_copy(emb_out_hbm.at[neg_idx], neg_rows,
                                  sem_g[b]).wait()
            pltpu.make_async_copy(emb_out_hbm.at[tgt_idx], tgt_rows,
                                  sem_g[b]).wait()

        def drain_scores(b):
            scores_v = bufs[b][6]
            pltpu.make_async_copy(scores_v, sc_hbm.at[pl.ds(0, C * S)],
                                  sem_o[b]).wait()

        def compute(ch, b):
            ctx_rows, neg_rows, tgt_rows, scores_v = bufs[b][3:7]
            base = wbase + ch * C

            def row_body(i, rc):
                rb = i * W
                acc = [ctx_rows[rb, pl.ds(c * 16, 16)] for c in range(4)]
                for j in range(1, W):
                    for c in range(4):
                        acc[c] = acc[c] + ctx_rows[rb + j, pl.ds(c * 16, 16)]
                nv0 = zeros
                nv1 = zeros
                p = acc[0] * tgt_rows[i, pl.ds(0, 16)]
                for c in range(1, 4):
                    p = p + acc[c] * tgt_rows[i, pl.ds(c * 16, 16)]
                nv1 = jnp.where(lane == (K - 16), lanesum(p), nv1)
                for k in range(K):
                    q = acc[0] * neg_rows[rb + k, pl.ds(0, 16)]
                    for c in range(1, 4):
                        q = q + acc[c] * neg_rows[rb + k, pl.ds(c * 16, 16)]
                    cq = lanesum(q)
                    if k < 16:
                        nv0 = jnp.where(lane == k, cq, nv0)
                    else:
                        nv1 = jnp.where(lane == (k - 16), cq, nv1)
                scores_v[pl.ds(i * S, 16)] = nv0
                scores_v[pl.ds(i * S + 16, 16)] = nv1
                return rc

            drain_scores(b)
            lax.fori_loop(0, C, row_body, 0)
            pltpu.async_copy(scores_v, sc_hbm.at[pl.ds(base * S, C * S)],
                             sem_o[b])

        issue_idx(jnp.int32(0), 0)
        drain_idx(0)
        issue_gathers(0)
        issue_idx(jnp.int32(1), 1)
        pltpu.async_copy(bufs[0][6], sc_hbm.at[pl.ds(wbase * S, C * S)],
                         sem_o[0])
        pltpu.async_copy(bufs[1][6], sc_hbm.at[pl.ds((wbase + C) * S, C * S)],
                         sem_o[1])

        last = jnp.int32(NCH - 1)

        def pair_body(i, carry):
            ch0 = i * 2
            ch1 = ch0 + 1
            drain_idx(1)
            issue_gathers(1)
            drain_gathers(0)
            issue_idx(jnp.minimum(ch0 + 2, last), 0)
            compute(ch0, 0)
            drain_idx(0)
            issue_gathers(0)
            drain_gathers(1)
            issue_idx(jnp.minimum(ch1 + 2, last), 1)
            compute(ch1, 1)
            return carry

        lax.fori_loop(0, NCH // 2, pair_body, 0)

        drain_gathers(0)
        drain_idx(1)
        drain_scores(0)
        drain_scores(1)

    return scores_kernel(emb_in, emb_out, ctx1d, tgt, neg1d)


def _transpose_body(x_ref, o_ref):
    o_ref[...] = x_ref[...].T


def _tc_transpose(xt, bn):
    v = xt.shape[1]
    grid = (v + bn - 1) // bn
    return pl.pallas_call(
        _transpose_body,
        grid=(grid,),
        in_specs=[pl.BlockSpec((D, bn), lambda i: (0, i))],
        out_specs=pl.BlockSpec((bn, D), lambda i: (i, 0)),
        out_shape=jax.ShapeDtypeStruct((v, D), jnp.float32),
    )(xt)


def _loss_body(sc_ref, out_ref):
    x = sc_ref[...]
    l32 = lax.broadcasted_iota(jnp.int32, x.shape, 1) % S
    soft_abs = jnp.log1p(jnp.exp(-jnp.abs(x)))
    sp_pos = jnp.maximum(x, 0.0) + soft_abs
    sp_neg = jnp.maximum(-x, 0.0) + soft_abs
    n_sum = jnp.sum(jnp.where(l32 < K, sp_pos, 0.0))
    t_sum = jnp.sum(jnp.where(l32 == K, sp_neg, 0.0))
    out_ref[0, 0] = t_sum / B + n_sum / (B * K)


def kernel(context, target, negative_samples, emb_in, emb_out):
    ctx1d = context.reshape(B * W).astype(jnp.int32)
    neg1d = negative_samples.reshape(B * K).astype(jnp.int32)
    tgt = target.astype(jnp.int32)

    scores = _sc_scores(emb_in, emb_out, ctx1d, tgt, neg1d)

    loss = pl.pallas_call(
        _loss_body,
        out_shape=jax.ShapeDtypeStruct((1, 1), jnp.float32),
        in_specs=[pl.BlockSpec(memory_space=pltpu.VMEM)],
        out_specs=pl.BlockSpec(memory_space=pltpu.SMEM),
    )(scores.reshape(B * S // 128, 128))
    return loss[0, 0]


# speedup vs baseline: 1.7021x; 1.1131x over previous
import functools

import jax
import jax.numpy as jnp
from jax import lax
from jax.experimental import pallas as pl
from jax.experimental.pallas import tpu as pltpu
from jax.experimental.pallas import tpu_sc as plsc

B = 16384
W = 20
K = 20
D = 64
D2 = 128
NW = 32
ROWS_PER_W = B // NW
C = 8
NCH = ROWS_PER_W // C
S = 32


def _sc_scores(emb_cat, ctx1d, tgt, neg1d):
    mesh = plsc.VectorSubcoreMesh(core_axis_name="c", subcore_axis_name="s")

    buf_types = []
    for _ in range(2):
        buf_types += [
            pltpu.VMEM((C * W,), jnp.int32),
            pltpu.VMEM((C * K,), jnp.int32),
            pltpu.VMEM((C,), jnp.int32),
            pltpu.VMEM((C * W, D2), jnp.float32),
            pltpu.VMEM((C * K, D2), jnp.float32),
            pltpu.VMEM((C, D2), jnp.float32),
            pltpu.VMEM((C * S,), jnp.float32),
        ]
    sem_types = [pltpu.SemaphoreType.DMA] * 6

    @functools.partial(
        pl.kernel,
        out_type=jax.ShapeDtypeStruct((B * S,), jnp.float32),
        mesh=mesh,
        scratch_types=buf_types + sem_types,
        compiler_params=pltpu.CompilerParams(use_tc_tiling_on_sc=False),
    )
    def scores_kernel(emb_hbm, ctx_hbm, tgt_hbm, neg_hbm, sc_hbm, *scratch):
        bufs = [scratch[0:7], scratch[7:14]]
        sem_idx = scratch[14:16]
        sem_g = scratch[16:18]
        sem_o = scratch[18:20]

        cid = lax.axis_index("c")
        sid = lax.axis_index("s")
        wid = sid * 2 + cid
        wbase = wid * ROWS_PER_W

        lane = lax.iota(jnp.int32, 16)
        perms = [lane ^ s for s in (8, 4, 2, 1)]
        zeros = jnp.zeros((16,), jnp.float32)
        gdn = lax.GatherDimensionNumbers(
            offset_dims=(), collapsed_slice_dims=(0,), start_index_map=(0,))

        def lanesum(v):
            for p16 in perms:
                v = v + lax.gather(
                    v, p16[:, None], gdn, (1,),
                    mode=lax.GatherScatterMode.PROMISE_IN_BOUNDS)
            return v

        def issue_idx(ch, b):
            ctx_idx, neg_idx, tgt_idx = bufs[b][0:3]
            base = wbase + ch * C
            ibase = base * W
            pltpu.async_copy(ctx_hbm.at[pl.ds(ibase, C * W)], ctx_idx,
                             sem_idx[b])
            pltpu.async_copy(neg_hbm.at[pl.ds(ibase, C * K)], neg_idx,
                             sem_idx[b])
            pltpu.async_copy(tgt_hbm.at[pl.ds(base, C)], tgt_idx, sem_idx[b])

        def drain_idx(b):
            ctx_idx, neg_idx, tgt_idx = bufs[b][0:3]
            pltpu.make_async_copy(ctx_hbm.at[pl.ds(0, C * W)], ctx_idx,
                                  sem_idx[b]).wait()
            pltpu.make_async_copy(neg_hbm.at[pl.ds(0, C * K)], neg_idx,
                                  sem_idx[b]).wait()
            pltpu.make_async_copy(tgt_hbm.at[pl.ds(0, C)], tgt_idx,
                                  sem_idx[b]).wait()

        def issue_gathers(b):
            ctx_idx, neg_idx, tgt_idx, ctx_rows, neg_rows, tgt_rows = \
                bufs[b][0:6]
            pltpu.async_copy(emb_hbm.at[ctx_idx], ctx_rows, sem_g[b])
            pltpu.async_copy(emb_hbm.at[neg_idx], neg_rows, sem_g[b])
            pltpu.async_copy(emb_hbm.at[tgt_idx], tgt_rows, sem_g[b])

        def drain_gathers(b):
            ctx_idx, neg_idx, tgt_idx, ctx_rows, neg_rows, tgt_rows = \
                bufs[b][0:6]
            pltpu.make_async_copy(emb_hbm.at[ctx_idx], ctx_rows,
                                  sem_g[b]).wait()
            pltpu.make_async_copy(emb_hbm.at[neg_idx], neg_rows,
                                  sem_g[b]).wait()
            pltpu.make_async_copy(emb_hbm.at[tgt_idx], tgt_rows,
                                  sem_g[b]).wait()

        def drain_scores(b):
            scores_v = bufs[b][6]
            pltpu.make_async_copy(scores_v, sc_hbm.at[pl.ds(0, C * S)],
                                  sem_o[b]).wait()

        def compute(ch, b):
            ctx_rows, neg_rows, tgt_rows, scores_v = bufs[b][3:7]
            base = wbase + ch * C

            def row_body(i, rc):
                rb = i * W
                acc = [ctx_rows[rb, pl.ds(c * 16, 16)] for c in range(4)]
                for j in range(1, W):
                    for c in range(4):
                        acc[c] = acc[c] + ctx_rows[rb + j, pl.ds(c * 16, 16)]
                nv0 = zeros
                nv1 = zeros
                p = acc[0] * tgt_rows[i, pl.ds(D, 16)]
                for c in range(1, 4):
                    p = p + acc[c] * tgt_rows[i, pl.ds(D + c * 16, 16)]
                nv1 = jnp.where(lane == (K - 16), lanesum(p), nv1)
                for k in range(K):
                    q = acc[0] * neg_rows[rb + k, pl.ds(D, 16)]
                    for c in range(1, 4):
                        q = q + acc[c] * neg_rows[rb + k, pl.ds(D + c * 16, 16)]
                    cq = lanesum(q)
                    if k < 16:
                        nv0 = jnp.where(lane == k, cq, nv0)
                    else:
                        nv1 = jnp.where(lane == (k - 16), cq, nv1)
                scores_v[pl.ds(i * S, 16)] = nv0
                scores_v[pl.ds(i * S + 16, 16)] = nv1
                return rc

            drain_scores(b)
            lax.fori_loop(0, C, row_body, 0)
            pltpu.async_copy(scores_v, sc_hbm.at[pl.ds(base * S, C * S)],
                             sem_o[b])

        issue_idx(jnp.int32(0), 0)
        drain_idx(0)
        issue_gathers(0)
        issue_idx(jnp.int32(1), 1)
        pltpu.async_copy(bufs[0][6], sc_hbm.at[pl.ds(wbase * S, C * S)],
                         sem_o[0])
        pltpu.async_copy(bufs[1][6], sc_hbm.at[pl.ds((wbase + C) * S, C * S)],
                         sem_o[1])

        last = jnp.int32(NCH - 1)

        def pair_body(i, carry):
            ch0 = i * 2
            ch1 = ch0 + 1
            drain_idx(1)
            issue_gathers(1)
            drain_gathers(0)
            issue_idx(jnp.minimum(ch0 + 2, last), 0)
            compute(ch0, 0)
            drain_idx(0)
            issue_gathers(0)
            drain_gathers(1)
            issue_idx(jnp.minimum(ch1 + 2, last), 1)
            compute(ch1, 1)
            return carry

        lax.fori_loop(0, NCH // 2, pair_body, 0)

        drain_gathers(0)
        drain_idx(1)
        drain_scores(0)
        drain_scores(1)

    return scores_kernel(emb_cat, ctx1d, tgt, neg1d)


def _loss_body(sc_ref, out_ref):
    x = sc_ref[...]
    l32 = lax.broadcasted_iota(jnp.int32, x.shape, 1) % S
    soft_abs = jnp.log1p(jnp.exp(-jnp.abs(x)))
    sp_pos = jnp.maximum(x, 0.0) + soft_abs
    sp_neg = jnp.maximum(-x, 0.0) + soft_abs
    n_sum = jnp.sum(jnp.where(l32 < K, sp_pos, 0.0))
    t_sum = jnp.sum(jnp.where(l32 == K, sp_neg, 0.0))
    out_ref[0, 0] = t_sum / B + n_sum / (B * K)


def kernel(context, target, negative_samples, emb_in, emb_out):
    ctx1d = context.reshape(B * W).astype(jnp.int32)
    neg1d = negative_samples.reshape(B * K).astype(jnp.int32)
    tgt = target.astype(jnp.int32)

    emb_cat = jnp.concatenate([emb_in, emb_out], axis=1)

    scores = _sc_scores(emb_cat, ctx1d, tgt, neg1d)

    loss = pl.pallas_call(
        _loss_body,
        out_shape=jax.ShapeDtypeStruct((1, 1), jnp.float32),
        in_specs=[pl.BlockSpec(memory_space=pltpu.VMEM)],
        out_specs=pl.BlockSpec(memory_space=pltpu.SMEM),
    )(scores.reshape(B * S // 128, 128))
    return loss[0, 0]
